# Initial kernel scaffold; baseline (speedup 1.0000x reference)
import jax
import jax.numpy as jnp
from jax.experimental import pallas as pl


def kernel(x_ts, x_lc, batch_ts, batch_lc, W_ts0, b_ts0, W_ts1, b_ts1, W_lc0, b_lc0, W_lc1, b_lc1, W_conv, b_conv, W_o0, b_o0, W_o1, b_o1, W_o2, b_o2, W_o3, b_o3, W_o4, b_o4):
    raise NotImplementedError("write your pallas kernel here")


# speedup vs baseline: 4.7540x; 4.7540x over previous
import functools

import jax
import jax.numpy as jnp
from jax.experimental import pallas as pl
from jax.experimental.pallas import tpu as pltpu

N = 8192
B = 16
K = 16
HID = 16
R = 128
C = 512
NCHUNK = N // C
BIG = 3.0e7
NEG = -3.0e38


def _elu(x):
    return jnp.where(x > 0, x, jnp.exp(jnp.minimum(x, 0.0)) - 1.0)


def _enc_kernel(x_ts, x_lc, wt0, bt0, wt1, bt1, wl0, bl0, wl1, bl1,
                a_m, b_m, bc, ts_enc, lc_enc, u_lc, v_lc, v_ts):
    def lin(x, w, b):
        return jax.lax.dot_general(x, w[...], (((1,), (1,)), ((), ())),
                                   preferred_element_type=jnp.float32, precision=jax.lax.Precision.HIGHEST) + b[...][None, :]
    te = _elu(lin(_elu(lin(x_ts[...], wt0, bt0)), wt1, bt1))
    le = _elu(lin(_elu(lin(x_lc[...], wl0, bl0)), wl1, bl1))
    ts_enc[...] = te
    lc_enc[...] = le
    u_lc[...] = lin(le, b_m, jnp.zeros((HID,), jnp.float32))
    v_lc[...] = lin(le, a_m, bc[...])
    v_ts[...] = lin(te, a_m, bc[...])


def _proj_kernel(f1, f2, a_m, b_m, bc, u_f1, v_f2):
    def lin(x, w, b):
        return jax.lax.dot_general(x, w[...], (((1,), (1,)), ((), ())),
                                   preferred_element_type=jnp.float32, precision=jax.lax.Precision.HIGHEST) + b[...][None, :]
    u_f1[...] = lin(f1[...], b_m, jnp.zeros((HID,), jnp.float32))
    v_f2[...] = lin(f2[...], a_m, bc[...])


def _conv_kernel(src, u, bsrc, dst_blk, v_blk, bdst_blk, out_blk, d_s):
    ones11 = jnp.ones((1, 1), jnp.float32)

    def t_col(x_r1):
        return jax.lax.dot_general(ones11, x_r1, (((1,), (1,)), ((), ())),
                                   preferred_element_type=jnp.float32, precision=jax.lax.Precision.HIGHEST)

    def t_row(x_1r):
        return jax.lax.dot_general(x_1r, ones11, (((0,), (0,)), ((), ())),
                                   preferred_element_type=jnp.float32, precision=jax.lax.Precision.HIGHEST)

    db = dst_blk[...]
    dn_row = t_col(jnp.sum(db * db, axis=1, keepdims=True))
    bd_row = t_col(bdst_blk[...])

    b_lo = bdst_blk[0, 0]
    b_hi = bdst_blk[R - 1, 0]
    bs_all = bsrc[...]
    start = jnp.sum((bs_all < b_lo).astype(jnp.int32))
    end = jnp.sum((bs_all <= b_hi).astype(jnp.int32))
    c0 = start // C
    c1 = (end + C - 1) // C

    def chunk(c):
        return pl.ds(pl.multiple_of(c * C, C), C)

    def fill(c, _):
        s = src[chunk(c), :]
        cross = jax.lax.dot_general(s, db, (((1,), (1,)), ((), ())),
                                    preferred_element_type=jnp.float32, precision=jax.lax.Precision.HIGHEST)
        d = jnp.sum(s * s, axis=1, keepdims=True) - 2.0 * cross + dn_row
        same = bsrc[chunk(c), :] == bd_row
        d_s[chunk(c), :] = jnp.where(same, d, jnp.inf)
        return 0

    jax.lax.fori_loop(c0, c1, fill, 0)

    rows0 = jax.lax.broadcasted_iota(jnp.int32, (C, 1), 0).astype(jnp.float32)

    def round_body(_, acc):
        def minp(c, m):
            return jnp.minimum(m, jnp.min(d_s[chunk(c), :], axis=0,
                                          keepdims=True))
        m = jax.lax.fori_loop(c0, c1, minp, jnp.full((1, R), jnp.inf))

        def argp(c, j):
            rows = rows0 + (c * C).astype(jnp.float32)
            cand = jnp.where(d_s[chunk(c), :] == m, rows, BIG)
            return jnp.minimum(j, jnp.min(cand, axis=0, keepdims=True))
        jsel = jax.lax.fori_loop(c0, c1, argp, jnp.full((1, R), BIG))

        def gath(c, us):
            d = d_s[chunk(c), :]
            rows = rows0 + (c * C).astype(jnp.float32)
            oh = (d == m) & (rows == jsel)
            d_s[chunk(c), :] = jnp.where(oh, jnp.inf, d)
            ohf = oh.astype(jnp.float32)
            return us + jax.lax.dot_general(
                ohf, u[chunk(c), :], (((0,), (0,)), ((), ())),
                preferred_element_type=jnp.float32, precision=jax.lax.Precision.HIGHEST)
        u_sel = jax.lax.fori_loop(c0, c1, gath, jnp.zeros((R, HID),
                                                         jnp.float32))
        valid = t_row((jsel < BIG).astype(jnp.float32)) > 0.5
        msg = jnp.where(valid, _elu(v_blk[...] + u_sel), NEG)
        return jnp.maximum(acc, msg)

    out_blk[...] = jax.lax.fori_loop(0, K, round_body,
                                     jnp.full((R, HID), NEG, jnp.float32))


def _head_kernel(f3, bts, w0, b0, w1, b1, w2, b2, w3, b3, w4, b4, out):
    def lin(x, w, b):
        return jax.lax.dot_general(x, w[...], (((1,), (1,)), ((), ())),
                                   preferred_element_type=jnp.float32, precision=jax.lax.Precision.HIGHEST) + b[...][None, :]
    gid = jax.lax.broadcasted_iota(jnp.int32, (1, B), 1).astype(jnp.float32)
    oh = (bts[...] == gid).astype(jnp.float32)
    pooled = jax.lax.dot_general(oh, f3[...], (((0,), (0,)), ((), ())),
                                 preferred_element_type=jnp.float32, precision=jax.lax.Precision.HIGHEST)
    cnt = jnp.sum(oh, axis=0)[:, None]
    pooled = pooled / jnp.maximum(cnt, 1.0)
    h = _elu(lin(pooled, w0, b0))
    h = _elu(lin(h, w1, b1))
    h = _elu(lin(h, w2, b2))
    h = _elu(lin(h, w3, b3))
    o8 = jax.lax.dot_general(h, w4[...], (((1,), (1,)), ((), ())),
                             preferred_element_type=jnp.float32, precision=jax.lax.Precision.HIGHEST) + b4[0]
    out[...] = o8[:, 0:1]


def _full(shape):
    return pl.BlockSpec(shape, lambda *_: tuple(0 for _ in shape))


def _edge_conv(src, u, bsrc, dst, v, bdst):
    return pl.pallas_call(
        _conv_kernel,
        grid=(N // R,),
        in_specs=[
            _full((N, HID)),
            _full((N, HID)),
            _full((N, 1)),
            pl.BlockSpec((R, HID), lambda g: (g, 0)),
            pl.BlockSpec((R, HID), lambda g: (g, 0)),
            pl.BlockSpec((R, 1), lambda g: (g, 0)),
        ],
        out_specs=pl.BlockSpec((R, HID), lambda g: (g, 0)),
        out_shape=jax.ShapeDtypeStruct((N, HID), jnp.float32),
        scratch_shapes=[pltpu.VMEM((N, R), jnp.float32)],
    )(src, u, bsrc, dst, v, bdst)


def kernel(x_ts, x_lc, batch_ts, batch_lc, W_ts0, b_ts0, W_ts1, b_ts1,
           W_lc0, b_lc0, W_lc1, b_lc1, W_conv, b_conv, W_o0, b_o0,
           W_o1, b_o1, W_o2, b_o2, W_o3, b_o3, W_o4, b_o4):
    A = W_conv[:, :HID] - W_conv[:, HID:]
    Bm = W_conv[:, HID:]
    bts = batch_ts.astype(jnp.float32)[:, None]
    blc = batch_lc.astype(jnp.float32)[:, None]

    enc_out = pl.pallas_call(
        _enc_kernel,
        in_specs=[_full((N, 6)), _full((N, 5)),
                  _full((HID, 6)), _full((HID,)), _full((HID, HID)),
                  _full((HID,)), _full((HID, 5)), _full((HID,)),
                  _full((HID, HID)), _full((HID,)),
                  _full((HID, HID)), _full((HID, HID)), _full((HID,))],
        out_specs=[_full((N, HID))] * 5,
        out_shape=[jax.ShapeDtypeStruct((N, HID), jnp.float32)] * 5,
    )(x_ts, x_lc, W_ts0, b_ts0, W_ts1, b_ts1, W_lc0, b_lc0, W_lc1, b_lc1,
      A, Bm, b_conv)
    ts_enc, lc_enc, u_lc, v_lc, v_ts = enc_out

    f1 = _edge_conv(lc_enc, u_lc, blc, lc_enc, v_lc, blc)
    f2 = _edge_conv(lc_enc, u_lc, blc, ts_enc, v_ts, bts)

    u_f1, v_f2 = pl.pallas_call(
        _proj_kernel,
        in_specs=[_full((N, HID)), _full((N, HID)),
                  _full((HID, HID)), _full((HID, HID)), _full((HID,))],
        out_specs=[_full((N, HID))] * 2,
        out_shape=[jax.ShapeDtypeStruct((N, HID), jnp.float32)] * 2,
    )(f1, f2, A, Bm, b_conv)

    f3 = _edge_conv(f1, u_f1, blc, f2, v_f2, bts)

    out = pl.pallas_call(
        _head_kernel,
        in_specs=[_full((N, HID)), _full((N, 1)),
                  _full((64, HID)), _full((64,)), _full((32, 64)),
                  _full((32,)), _full((8, 32)), _full((8,)),
                  _full((4, 8)), _full((4,)), _full((8, 4)), _full((1,))],
        out_specs=_full((B, 1)),
        out_shape=jax.ShapeDtypeStruct((B, 1), jnp.float32),
    )(f3, bts, W_o0, b_o0, W_o1, b_o1, W_o2, b_o2, W_o3, b_o3,
      jnp.pad(W_o4, ((0, 7), (0, 0))), b_o4)

    return out, jnp.arange(B, dtype=jnp.int32)


# speedup vs baseline: 4.7608x; 1.0014x over previous
import functools

import jax
import jax.numpy as jnp
from jax.experimental import pallas as pl
from jax.experimental.pallas import tpu as pltpu

N = 8192
B = 16
K = 16
HID = 16
R = 128
C = 256
NCHUNK = N // C
BIGF = 1.0e30
NEG = -3.0e38


def _elu(x):
    return jnp.where(x > 0, x, jnp.exp(jnp.minimum(x, 0.0)) - 1.0)


def _enc_kernel(x_ts, x_lc, wt0, bt0, wt1, bt1, wl0, bl0, wl1, bl1,
                a_m, b_m, bc, ts_enc, lc_enc, u_lc, v_lc, v_ts):
    def lin(x, w, b):
        return jax.lax.dot_general(x, w[...], (((1,), (1,)), ((), ())),
                                   preferred_element_type=jnp.float32, precision=jax.lax.Precision.HIGHEST) + b[...][None, :]
    te = _elu(lin(_elu(lin(x_ts[...], wt0, bt0)), wt1, bt1))
    le = _elu(lin(_elu(lin(x_lc[...], wl0, bl0)), wl1, bl1))
    ts_enc[...] = te
    lc_enc[...] = le
    u_lc[...] = lin(le, b_m, jnp.zeros((HID,), jnp.float32))
    v_lc[...] = lin(le, a_m, bc[...])
    v_ts[...] = lin(te, a_m, bc[...])


def _proj_kernel(f1, f2, a_m, b_m, bc, u_f1, v_f2):
    def lin(x, w, b):
        return jax.lax.dot_general(x, w[...], (((1,), (1,)), ((), ())),
                                   preferred_element_type=jnp.float32, precision=jax.lax.Precision.HIGHEST) + b[...][None, :]
    u_f1[...] = lin(f1[...], b_m, jnp.zeros((HID,), jnp.float32))
    v_f2[...] = lin(f2[...], a_m, bc[...])


def _conv_kernel(src, u, bsrc, dst_blk, v_blk, bdst_blk, out_blk, d_s):
    ones11 = jnp.ones((1, 1), jnp.float32)

    def t_col(x_r1):
        return jax.lax.dot_general(ones11, x_r1, (((1,), (1,)), ((), ())),
                                   preferred_element_type=jnp.float32, precision=jax.lax.Precision.HIGHEST)

    def t_row(x_1r):
        return jax.lax.dot_general(x_1r, ones11, (((0,), (0,)), ((), ())),
                                   preferred_element_type=jnp.float32, precision=jax.lax.Precision.HIGHEST)

    db = dst_blk[...]
    dn_row = t_col(jnp.sum(db * db, axis=1, keepdims=True))
    bd_row = t_col(bdst_blk[...])

    b_lo = bdst_blk[0, 0]
    b_hi = bdst_blk[R - 1, 0]
    bs_all = bsrc[...]
    start = jnp.sum((bs_all < b_lo).astype(jnp.int32))
    end = jnp.sum((bs_all <= b_hi).astype(jnp.int32))
    c0 = start // C
    c1 = (end + C - 1) // C

    def chunk(c):
        return pl.ds(pl.multiple_of(c * C, C), C)

    def fill(c, m):
        s = src[chunk(c), :]
        cross = jax.lax.dot_general(s, db, (((1,), (1,)), ((), ())),
                                    preferred_element_type=jnp.float32,
                                    precision=jax.lax.Precision.HIGHEST)
        d = jnp.sum(s * s, axis=1, keepdims=True) - 2.0 * cross + dn_row
        same = bsrc[chunk(c), :] == bd_row
        d = jnp.where(same, d, BIGF)
        d_s[chunk(c), :] = d
        return jnp.minimum(m, jnp.min(d, axis=0, keepdims=True))

    m0 = jax.lax.fori_loop(c0, c1, fill, jnp.full((1, R), jnp.inf))

    def round_body(_, carry):
        m, maxp = carry

        def sweep(c, st):
            mn, cnt, us = st
            d = d_s[chunk(c), :]
            oh = d == m
            dn = jnp.where(oh, jnp.inf, d)
            d_s[chunk(c), :] = dn
            ohf = oh.astype(jnp.float32)
            mn = jnp.minimum(mn, jnp.min(dn, axis=0, keepdims=True))
            cnt = cnt + jnp.sum(ohf, axis=0, keepdims=True)
            us = us + jax.lax.dot_general(
                ohf, u[chunk(c), :], (((0,), (0,)), ((), ())),
                preferred_element_type=jnp.float32,
                precision=jax.lax.Precision.HIGHEST)
            return mn, cnt, us

        mn, cnt, us = jax.lax.fori_loop(
            c0, c1, sweep,
            (jnp.full((1, R), jnp.inf), jnp.zeros((1, R), jnp.float32),
             jnp.zeros((R, HID), jnp.float32)))
        cnt_col = t_row(cnt)
        us = us * (1.0 / jnp.maximum(cnt_col, 1.0))
        maxp = jnp.maximum(maxp, jnp.where(cnt_col > 0.0, us, NEG))
        return mn, maxp

    _, maxp = jax.lax.fori_loop(
        0, K, round_body, (m0, jnp.full((R, HID), NEG, jnp.float32)))
    out_blk[...] = _elu(v_blk[...] + maxp)


def _head_kernel(f3, bts, w0, b0, w1, b1, w2, b2, w3, b3, w4, b4, out):
    def lin(x, w, b):
        return jax.lax.dot_general(x, w[...], (((1,), (1,)), ((), ())),
                                   preferred_element_type=jnp.float32, precision=jax.lax.Precision.HIGHEST) + b[...][None, :]
    gid = jax.lax.broadcasted_iota(jnp.int32, (1, B), 1).astype(jnp.float32)
    oh = (bts[...] == gid).astype(jnp.float32)
    pooled = jax.lax.dot_general(oh, f3[...], (((0,), (0,)), ((), ())),
                                 preferred_element_type=jnp.float32, precision=jax.lax.Precision.HIGHEST)
    cnt = jnp.sum(oh, axis=0)[:, None]
    pooled = pooled / jnp.maximum(cnt, 1.0)
    h = _elu(lin(pooled, w0, b0))
    h = _elu(lin(h, w1, b1))
    h = _elu(lin(h, w2, b2))
    h = _elu(lin(h, w3, b3))
    o8 = jax.lax.dot_general(h, w4[...], (((1,), (1,)), ((), ())),
                             preferred_element_type=jnp.float32, precision=jax.lax.Precision.HIGHEST) + b4[0]
    out[...] = o8[:, 0:1]


def _full(shape):
    return pl.BlockSpec(shape, lambda *_: tuple(0 for _ in shape))


def _edge_conv(src, u, bsrc, dst, v, bdst):
    return pl.pallas_call(
        _conv_kernel,
        grid=(N // R,),
        in_specs=[
            _full((N, HID)),
            _full((N, HID)),
            _full((N, 1)),
            pl.BlockSpec((R, HID), lambda g: (g, 0)),
            pl.BlockSpec((R, HID), lambda g: (g, 0)),
            pl.BlockSpec((R, 1), lambda g: (g, 0)),
        ],
        out_specs=pl.BlockSpec((R, HID), lambda g: (g, 0)),
        out_shape=jax.ShapeDtypeStruct((N, HID), jnp.float32),
        scratch_shapes=[pltpu.VMEM((N, R), jnp.float32)],
    )(src, u, bsrc, dst, v, bdst)


def kernel(x_ts, x_lc, batch_ts, batch_lc, W_ts0, b_ts0, W_ts1, b_ts1,
           W_lc0, b_lc0, W_lc1, b_lc1, W_conv, b_conv, W_o0, b_o0,
           W_o1, b_o1, W_o2, b_o2, W_o3, b_o3, W_o4, b_o4):
    A = W_conv[:, :HID] - W_conv[:, HID:]
    Bm = W_conv[:, HID:]
    bts = batch_ts.astype(jnp.float32)[:, None]
    blc = batch_lc.astype(jnp.float32)[:, None]

    enc_out = pl.pallas_call(
        _enc_kernel,
        in_specs=[_full((N, 6)), _full((N, 5)),
                  _full((HID, 6)), _full((HID,)), _full((HID, HID)),
                  _full((HID,)), _full((HID, 5)), _full((HID,)),
                  _full((HID, HID)), _full((HID,)),
                  _full((HID, HID)), _full((HID, HID)), _full((HID,))],
        out_specs=[_full((N, HID))] * 5,
        out_shape=[jax.ShapeDtypeStruct((N, HID), jnp.float32)] * 5,
    )(x_ts, x_lc, W_ts0, b_ts0, W_ts1, b_ts1, W_lc0, b_lc0, W_lc1, b_lc1,
      A, Bm, b_conv)
    ts_enc, lc_enc, u_lc, v_lc, v_ts = enc_out

    f1 = _edge_conv(lc_enc, u_lc, blc, lc_enc, v_lc, blc)
    f2 = _edge_conv(lc_enc, u_lc, blc, ts_enc, v_ts, bts)

    u_f1, v_f2 = pl.pallas_call(
        _proj_kernel,
        in_specs=[_full((N, HID)), _full((N, HID)),
                  _full((HID, HID)), _full((HID, HID)), _full((HID,))],
        out_specs=[_full((N, HID))] * 2,
        out_shape=[jax.ShapeDtypeStruct((N, HID), jnp.float32)] * 2,
    )(f1, f2, A, Bm, b_conv)

    f3 = _edge_conv(f1, u_f1, blc, f2, v_f2, bts)

    out = pl.pallas_call(
        _head_kernel,
        in_specs=[_full((N, HID)), _full((N, 1)),
                  _full((64, HID)), _full((64,)), _full((32, 64)),
                  _full((32,)), _full((8, 32)), _full((8,)),
                  _full((4, 8)), _full((4,)), _full((8, 4)), _full((1,))],
        out_specs=_full((B, 1)),
        out_shape=jax.ShapeDtypeStruct((B, 1), jnp.float32),
    )(f3, bts, W_o0, b_o0, W_o1, b_o1, W_o2, b_o2, W_o3, b_o3,
      jnp.pad(W_o4, ((0, 7), (0, 0))), b_o4)

    return out, jnp.arange(B, dtype=jnp.int32)


# speedup vs baseline: 5.0491x; 1.0605x over previous
import functools

import jax
import jax.numpy as jnp
from jax.experimental import pallas as pl
from jax.experimental.pallas import tpu as pltpu

N = 8192
B = 16
K = 16
HID = 16
R = 128
C = 256
NCHUNK = N // C
BIGF = 1.0e30
NEG = -3.0e38


def _elu(x):
    return jnp.where(x > 0, x, jnp.exp(jnp.minimum(x, 0.0)) - 1.0)


def _enc_kernel(x_ts, x_lc, wt0, bt0, wt1, bt1, wl0, bl0, wl1, bl1,
                a_m, b_m, bc, ts_enc, lc_enc, u_lc, v_lc, v_ts):
    def lin(x, w, b):
        return jax.lax.dot_general(x, w[...], (((1,), (1,)), ((), ())),
                                   preferred_element_type=jnp.float32, precision=jax.lax.Precision.HIGHEST) + b[...][None, :]
    te = _elu(lin(_elu(lin(x_ts[...], wt0, bt0)), wt1, bt1))
    le = _elu(lin(_elu(lin(x_lc[...], wl0, bl0)), wl1, bl1))
    ts_enc[...] = te
    lc_enc[...] = le
    u_lc[...] = lin(le, b_m, jnp.zeros((HID,), jnp.float32))
    v_lc[...] = lin(le, a_m, bc[...])
    v_ts[...] = lin(te, a_m, bc[...])


def _proj_kernel(f1, f2, a_m, b_m, bc, u_f1, v_f2):
    def lin(x, w, b):
        return jax.lax.dot_general(x, w[...], (((1,), (1,)), ((), ())),
                                   preferred_element_type=jnp.float32, precision=jax.lax.Precision.HIGHEST) + b[...][None, :]
    u_f1[...] = lin(f1[...], b_m, jnp.zeros((HID,), jnp.float32))
    v_f2[...] = lin(f2[...], a_m, bc[...])


def _conv_kernel(src, u, bsrc, bsrc_sq, dst_blk, v_blk, bdst_blk, out_blk,
                 d_s):
    ones11 = jnp.ones((1, 1), jnp.float32)

    def t_col(x_r1):
        return jax.lax.dot_general(ones11, x_r1, (((1,), (1,)), ((), ())),
                                   preferred_element_type=jnp.float32, precision=jax.lax.Precision.HIGHEST)

    def t_row(x_1r):
        return jax.lax.dot_general(x_1r, ones11, (((0,), (0,)), ((), ())),
                                   preferred_element_type=jnp.float32, precision=jax.lax.Precision.HIGHEST)

    db = dst_blk[...]
    dn_row = t_col(jnp.sum(db * db, axis=1, keepdims=True))
    bd_row = t_col(bdst_blk[...])

    b_lo = bdst_blk[0, 0]
    b_hi = bdst_blk[R - 1, 0]
    bsq = bsrc_sq[...]
    start = jnp.sum((bsq < b_lo).astype(jnp.int32))
    end = jnp.sum((bsq <= b_hi).astype(jnp.int32))
    c0 = start // C
    c1 = (end + C - 1) // C

    def chunk(c):
        return pl.ds(pl.multiple_of(c * C, C), C)

    def fill(c, m):
        s = src[chunk(c), :]
        cross = jax.lax.dot_general(s, db, (((1,), (1,)), ((), ())),
                                    preferred_element_type=jnp.float32,
                                    precision=jax.lax.Precision.HIGHEST)
        d = jnp.sum(s * s, axis=1, keepdims=True) - 2.0 * cross + dn_row
        same = bsrc[chunk(c), :] == bd_row
        d = jnp.where(same, d, BIGF)
        d_s[chunk(c), :] = d
        return jnp.minimum(m, jnp.min(d, axis=0, keepdims=True))

    m0 = jax.lax.fori_loop(c0, c1, fill, jnp.full((1, R), jnp.inf))

    def round_body(_, carry):
        m, maxp = carry

        def sweep(c, st):
            mn, cnt, us = st
            d = d_s[chunk(c), :]
            oh = d == m
            dn = jnp.where(oh, jnp.inf, d)
            d_s[chunk(c), :] = dn
            ohf = oh.astype(jnp.float32)
            mn = jnp.minimum(mn, jnp.min(dn, axis=0, keepdims=True))
            cnt = cnt + jnp.sum(ohf, axis=0, keepdims=True)
            us = us + jax.lax.dot_general(
                ohf, u[chunk(c), :], (((0,), (0,)), ((), ())),
                preferred_element_type=jnp.float32,
                precision=jax.lax.Precision.HIGHEST)
            return mn, cnt, us

        mn, cnt, us = jax.lax.fori_loop(
            c0, c1, sweep,
            (jnp.full((1, R), jnp.inf), jnp.zeros((1, R), jnp.float32),
             jnp.zeros((R, HID), jnp.float32)))
        cnt_col = t_row(cnt)
        us = us * (1.0 / jnp.maximum(cnt_col, 1.0))
        maxp = jnp.maximum(maxp, jnp.where(cnt_col > 0.0, us, NEG))
        return mn, maxp

    _, maxp = jax.lax.fori_loop(
        0, K, round_body, (m0, jnp.full((R, HID), NEG, jnp.float32)))
    out_blk[...] = _elu(v_blk[...] + maxp)


def _head_kernel(f3, bts, w0, b0, w1, b1, w2, b2, w3, b3, w4, b4, out):
    def lin(x, w, b):
        return jax.lax.dot_general(x, w[...], (((1,), (1,)), ((), ())),
                                   preferred_element_type=jnp.float32, precision=jax.lax.Precision.HIGHEST) + b[...][None, :]
    gid = jax.lax.broadcasted_iota(jnp.int32, (1, B), 1).astype(jnp.float32)
    oh = (bts[...] == gid).astype(jnp.float32)
    pooled = jax.lax.dot_general(oh, f3[...], (((0,), (0,)), ((), ())),
                                 preferred_element_type=jnp.float32, precision=jax.lax.Precision.HIGHEST)
    cnt = jnp.sum(oh, axis=0)[:, None]
    pooled = pooled / jnp.maximum(cnt, 1.0)
    h = _elu(lin(pooled, w0, b0))
    h = _elu(lin(h, w1, b1))
    h = _elu(lin(h, w2, b2))
    h = _elu(lin(h, w3, b3))
    o8 = jax.lax.dot_general(h, w4[...], (((1,), (1,)), ((), ())),
                             preferred_element_type=jnp.float32, precision=jax.lax.Precision.HIGHEST) + b4[0]
    out[...] = o8[:, 0:1]


def _full(shape):
    return pl.BlockSpec(shape, lambda *_: tuple(0 for _ in shape))


def _edge_conv(src, u, bsrc, bsrc_sq, dst, v, bdst):
    return pl.pallas_call(
        _conv_kernel,
        grid=(N // R,),
        in_specs=[
            _full((N, HID)),
            _full((N, HID)),
            _full((N, 1)),
            _full((N // 128, 128)),
            pl.BlockSpec((R, HID), lambda g: (g, 0)),
            pl.BlockSpec((R, HID), lambda g: (g, 0)),
            pl.BlockSpec((R, 1), lambda g: (g, 0)),
        ],
        out_specs=pl.BlockSpec((R, HID), lambda g: (g, 0)),
        out_shape=jax.ShapeDtypeStruct((N, HID), jnp.float32),
        scratch_shapes=[pltpu.VMEM((N, R), jnp.float32)],
    )(src, u, bsrc, bsrc_sq, dst, v, bdst)


def kernel(x_ts, x_lc, batch_ts, batch_lc, W_ts0, b_ts0, W_ts1, b_ts1,
           W_lc0, b_lc0, W_lc1, b_lc1, W_conv, b_conv, W_o0, b_o0,
           W_o1, b_o1, W_o2, b_o2, W_o3, b_o3, W_o4, b_o4):
    A = W_conv[:, :HID] - W_conv[:, HID:]
    Bm = W_conv[:, HID:]
    bts = batch_ts.astype(jnp.float32)[:, None]
    blc = batch_lc.astype(jnp.float32)[:, None]
    blc_sq = batch_lc.astype(jnp.float32).reshape(N // 128, 128)

    enc_out = pl.pallas_call(
        _enc_kernel,
        in_specs=[_full((N, 6)), _full((N, 5)),
                  _full((HID, 6)), _full((HID,)), _full((HID, HID)),
                  _full((HID,)), _full((HID, 5)), _full((HID,)),
                  _full((HID, HID)), _full((HID,)),
                  _full((HID, HID)), _full((HID, HID)), _full((HID,))],
        out_specs=[_full((N, HID))] * 5,
        out_shape=[jax.ShapeDtypeStruct((N, HID), jnp.float32)] * 5,
    )(x_ts, x_lc, W_ts0, b_ts0, W_ts1, b_ts1, W_lc0, b_lc0, W_lc1, b_lc1,
      A, Bm, b_conv)
    ts_enc, lc_enc, u_lc, v_lc, v_ts = enc_out

    f1 = _edge_conv(lc_enc, u_lc, blc, blc_sq, lc_enc, v_lc, blc)
    f2 = _edge_conv(lc_enc, u_lc, blc, blc_sq, ts_enc, v_ts, bts)

    u_f1, v_f2 = pl.pallas_call(
        _proj_kernel,
        in_specs=[_full((N, HID)), _full((N, HID)),
                  _full((HID, HID)), _full((HID, HID)), _full((HID,))],
        out_specs=[_full((N, HID))] * 2,
        out_shape=[jax.ShapeDtypeStruct((N, HID), jnp.float32)] * 2,
    )(f1, f2, A, Bm, b_conv)

    f3 = _edge_conv(f1, u_f1, blc, blc_sq, f2, v_f2, bts)

    out = pl.pallas_call(
        _head_kernel,
        in_specs=[_full((N, HID)), _full((N, 1)),
                  _full((64, HID)), _full((64,)), _full((32, 64)),
                  _full((32,)), _full((8, 32)), _full((8,)),
                  _full((4, 8)), _full((4,)), _full((8, 4)), _full((1,))],
        out_specs=_full((B, 1)),
        out_shape=jax.ShapeDtypeStruct((B, 1), jnp.float32),
    )(f3, bts, W_o0, b_o0, W_o1, b_o1, W_o2, b_o2, W_o3, b_o3,
      jnp.pad(W_o4, ((0, 7), (0, 0))), b_o4)

    return out, jnp.arange(B, dtype=jnp.int32)


# speedup vs baseline: 18.4514x; 3.6544x over previous
import functools

import jax
import jax.numpy as jnp
from jax import lax
from jax.experimental import pallas as pl
from jax.experimental.pallas import tpu as pltpu
from jax.experimental.pallas import tpu_sc as plsc

N = 8192
B = 16
K = 16
HID = 16
RS = 256
W = 2048
NEG = -3.0e38
PREC = jax.lax.Precision.HIGHEST


def _elu(x):
    return jnp.where(x > 0, x, jnp.exp(jnp.minimum(x, 0.0)) - 1.0)


def _enc_kernel(x_ts, x_lc, wt0, bt0, wt1, bt1, wl0, bl0, wl1, bl1,
                a_m, b_m, bc, ts_enc, lc_enc, u_lc, v_lc, v_ts, lcT):
    def lin(x, w, b):
        return jax.lax.dot_general(x, w[...], (((1,), (1,)), ((), ())),
                                   preferred_element_type=jnp.float32,
                                   precision=PREC) + b[...][None, :]
    te = _elu(lin(_elu(lin(x_ts[...], wt0, bt0)), wt1, bt1))
    le = _elu(lin(_elu(lin(x_lc[...], wl0, bl0)), wl1, bl1))
    ts_enc[...] = te
    lc_enc[...] = le
    u_lc[...] = lin(le, b_m, jnp.zeros((HID,), jnp.float32))
    v_lc[...] = lin(le, a_m, bc[...])
    v_ts[...] = lin(te, a_m, bc[...])
    eye = jnp.eye(HID, dtype=jnp.float32)
    lcT[...] = jax.lax.dot_general(eye, le, (((1,), (1,)), ((), ())),
                                   preferred_element_type=jnp.float32,
                                   precision=PREC)


def _seg_kernel(lc_enc, blc, sl, el, sn_lc):
    gid = jax.lax.broadcasted_iota(jnp.int32, (1, B), 1).astype(jnp.float32)
    oh = (blc[...] == gid).astype(jnp.float32)
    counts = jnp.sum(oh, axis=0, keepdims=True)
    r_i = jax.lax.broadcasted_iota(jnp.int32, (B, B), 0)
    c_i = jax.lax.broadcasted_iota(jnp.int32, (B, B), 1)
    strict_lower = (r_i < c_i).astype(jnp.float32)
    starts = jax.lax.dot_general(counts, strict_lower,
                                 (((1,), (0,)), ((), ())),
                                 preferred_element_type=jnp.float32,
                                 precision=PREC)
    sl[...] = starts
    el[...] = starts + counts
    le = lc_enc[...]
    sn_lc[...] = jnp.sum(le * le, axis=1, keepdims=True)


def _proj_kernel(f1, f2, a_m, b_m, bc, u_f1, v_f2, f1T, sn_f1):
    def lin(x, w, b):
        return jax.lax.dot_general(x, w[...], (((1,), (1,)), ((), ())),
                                   preferred_element_type=jnp.float32,
                                   precision=PREC) + b[...][None, :]
    u_f1[...] = lin(f1[...], b_m, jnp.zeros((HID,), jnp.float32))
    v_f2[...] = lin(f2[...], a_m, bc[...])
    eye = jnp.eye(HID, dtype=jnp.float32)
    f1T[...] = jax.lax.dot_general(eye, f1[...], (((1,), (1,)), ((), ())),
                                   preferred_element_type=jnp.float32,
                                   precision=PREC)
    sn_f1[...] = jnp.sum(f1[...] * f1[...], axis=1, keepdims=True)


def _sc_conv_body(srcT_hbm, sn_hbm, u_hbm, dst_hbm, v_hbm, bdst_hbm, sl_hbm,
                  el_hbm, out_hbm, st_win, sn_win, u_win, dstf, vf, outf,
                  bdstv, startsv, endsv, tstore, tistore, mpstore):
    wid = lax.axis_index("s") * 2 + lax.axis_index("c")
    base = wid * RS
    iota = lax.iota(jnp.int32, 16)

    pltpu.sync_copy(dst_hbm.at[pl.ds(base * HID, RS * HID)], dstf)
    pltpu.sync_copy(v_hbm.at[pl.ds(base * HID, RS * HID)], vf)
    pltpu.sync_copy(bdst_hbm.at[pl.ds(base, RS)], bdstv)
    pltpu.sync_copy(sl_hbm, startsv)
    pltpu.sync_copy(el_hbm, endsv)

    def init_body(i, _):
        tstore[pl.ds(i * 16, 16)] = jnp.full((16,), jnp.inf, jnp.float32)
        tistore[pl.ds(i * 16, 16)] = jnp.zeros((16,), jnp.int32)
        mpstore[pl.ds(i * 16, 16)] = jnp.full((16,), NEG, jnp.float32)
        return 0
    lax.fori_loop(0, RS, init_body, 0)

    svec = startsv[...].astype(jnp.int32)
    evec = endsv[...].astype(jnp.int32)

    def pick_i(vec, lane):
        return jnp.sum(jnp.where(iota == lane, vec.astype(jnp.float32),
                                 0.0)).astype(jnp.int32)

    bv0 = bdstv[pl.ds(0, 16)]
    bvl = bdstv[pl.ds(RS - 16, 16)]
    b_first = pick_i(bv0, 0)
    b_last = pick_i(bvl, 15)
    w_lo = (pick_i(svec, b_first) // 8) * 8
    w_hi = pick_i(evec, b_last)
    nwin = (w_hi - w_lo + W - 1) // W

    def win_body(wi, _):
        w0 = w_lo + wi * W
        w0c = jnp.minimum(w0, N - W)
        for f in range(HID):
            pltpu.sync_copy(srcT_hbm.at[pl.ds(f * N + w0c, W)],
                            st_win.at[pl.ds(f * W, W)])
        pltpu.sync_copy(sn_hbm.at[pl.ds(w0c, W)], sn_win)
        wcov_hi = jnp.minimum(w0 + W, N)

        def dst_body(i, _):
            xi = dstf[pl.ds(i * 16, 16)]
            bv = bdstv[pl.ds((i // 16) * 16, 16)]
            b_i = pick_i(bv, i - (i // 16) * 16)
            s_i = pick_i(svec, b_i)
            e_i = pick_i(evec, b_i)
            cs = jnp.maximum(s_i, w0)
            ce = jnp.minimum(e_i, wcov_hi)
            nch = jnp.maximum(ce - cs + 15, 0) // 16
            t = tstore[pl.ds(i * 16, 16)]
            ti = tistore[pl.ds(i * 16, 16)]
            xif = [jnp.sum(jnp.where(iota == f, xi, 0.0))
                   for f in range(HID)]
            xn = jnp.sum(xi * xi)
            t15 = jnp.max(t)

            def ch_body(c, carry):
                t, ti, t15 = carry
                j0 = cs + c * 16
                off = j0 - w0c
                prods = [st_win[pl.ds(f * W + off, 16)] * xif[f]
                         for f in range(HID)]
                while len(prods) > 1:
                    prods = [prods[a] + prods[a + 1]
                             for a in range(0, len(prods), 2)]
                cross = prods[0]
                d = (xn + sn_win[pl.ds(off, 16)]) - 2.0 * cross
                d = jnp.where(iota < (ce - j0), d, jnp.inf)
                ci = j0 + iota
                dmin = jnp.min(d)

                def mrg(tt):
                    t_, ti_ = tt
                    ds_, is_ = plsc.sort_key_val(d, ci)
                    dsr = lax.rev(ds_, (0,))
                    isr = lax.rev(is_, (0,))
                    keep = (t_ < dsr) | ((t_ == dsr) & (ti_ < isr))
                    lo_v = jnp.where(keep, t_, dsr)
                    lo_i = jnp.where(keep, ti_, isr)
                    t2, ti2 = plsc.sort_key_val(lo_v, lo_i)
                    return t2, ti2

                t, ti = lax.cond(dmin < t15, mrg, lambda tt: tt, (t, ti))
                t15 = jnp.max(t)
                return t, ti, t15

            t, ti, _ = lax.fori_loop(0, nch, ch_body, (t, ti, t15))
            tstore[pl.ds(i * 16, 16)] = t
            tistore[pl.ds(i * 16, 16)] = ti
            return 0

        lax.fori_loop(0, RS, dst_body, 0)
        return 0

    lax.fori_loop(0, nwin, win_body, 0)

    def upool_body(wi, _):
        w0 = w_lo + wi * W
        w0c = jnp.minimum(w0, N - W)
        pltpu.sync_copy(u_hbm.at[pl.ds(w0c * HID, W * HID)], u_win)

        def dst_body(i, _):
            ti = tistore[pl.ds(i * 16, 16)]
            inm = (ti >= w0) & (ti < w0 + W)
            til = jnp.where(inm, ti - w0c, 0) * HID
            mp = mpstore[pl.ds(i * 16, 16)]
            for h in range(HID):
                g = plsc.load_gather(u_win, [til + h])
                s_h = jnp.max(jnp.where(inm, g, NEG))
                mp = jnp.where(iota == h, jnp.maximum(mp, s_h), mp)
            mpstore[pl.ds(i * 16, 16)] = mp
            return 0

        lax.fori_loop(0, RS, dst_body, 0)
        return 0

    lax.fori_loop(0, nwin, upool_body, 0)

    def fin_body(i, _):
        x = vf[pl.ds(i * 16, 16)] + mpstore[pl.ds(i * 16, 16)]
        outf[pl.ds(i * 16, 16)] = jnp.where(
            x > 0, x, jnp.exp(jnp.minimum(x, 0.0)) - 1.0)
        return 0
    lax.fori_loop(0, RS, fin_body, 0)

    pltpu.sync_copy(outf, out_hbm.at[pl.ds(base * HID, RS * HID)])


def _sc_edge_conv(srcT_flat, sn_flat, u_flat, dst_flat, v_flat, bdst_i, sl_flat, el_flat):
    mesh = plsc.VectorSubcoreMesh(core_axis_name="c", subcore_axis_name="s")
    fn = pl.kernel(
        _sc_conv_body,
        out_type=pltpu.HBM((N * HID,), jnp.float32),
        mesh=mesh,
        compiler_params=pltpu.CompilerParams(needs_layout_passes=False),
        scratch_types=[
            pltpu.VMEM((HID * W,), jnp.float32),
            pltpu.VMEM((W,), jnp.float32),
            pltpu.VMEM((W * HID,), jnp.float32),
            pltpu.VMEM((RS * HID,), jnp.float32),
            pltpu.VMEM((RS * HID,), jnp.float32),
            pltpu.VMEM((RS * HID,), jnp.float32),
            pltpu.VMEM((RS,), jnp.int32),
            pltpu.VMEM((B,), jnp.float32),
            pltpu.VMEM((B,), jnp.float32),
            pltpu.VMEM((RS * 16,), jnp.float32),
            pltpu.VMEM((RS * 16,), jnp.int32),
            pltpu.VMEM((RS * 16,), jnp.float32),
        ],
    )
    return fn(srcT_flat, sn_flat, u_flat, dst_flat, v_flat, bdst_i, sl_flat, el_flat)


def _head_kernel(f3, bts, w0, b0, w1, b1, w2, b2, w3, b3, w4, b4, out):
    def lin(x, w, b):
        return jax.lax.dot_general(x, w[...], (((1,), (1,)), ((), ())),
                                   preferred_element_type=jnp.float32,
                                   precision=PREC) + b[...][None, :]
    gid = jax.lax.broadcasted_iota(jnp.int32, (1, B), 1).astype(jnp.float32)
    oh = (bts[...] == gid).astype(jnp.float32)
    pooled = jax.lax.dot_general(oh, f3[...], (((0,), (0,)), ((), ())),
                                 preferred_element_type=jnp.float32,
                                 precision=PREC)
    cnt = jnp.sum(oh, axis=0)[:, None]
    pooled = pooled / jnp.maximum(cnt, 1.0)
    h = _elu(lin(pooled, w0, b0))
    h = _elu(lin(h, w1, b1))
    h = _elu(lin(h, w2, b2))
    h = _elu(lin(h, w3, b3))
    o8 = jax.lax.dot_general(h, w4[...], (((1,), (1,)), ((), ())),
                             preferred_element_type=jnp.float32,
                             precision=PREC) + b4[0]
    out[...] = o8[:, 0:1]


def _full(shape):
    return pl.BlockSpec(shape, lambda *_: tuple(0 for _ in shape))


def kernel(x_ts, x_lc, batch_ts, batch_lc, W_ts0, b_ts0, W_ts1, b_ts1,
           W_lc0, b_lc0, W_lc1, b_lc1, W_conv, b_conv, W_o0, b_o0,
           W_o1, b_o1, W_o2, b_o2, W_o3, b_o3, W_o4, b_o4):
    A = W_conv[:, :HID] - W_conv[:, HID:]
    Bm = W_conv[:, HID:]
    bts = batch_ts.astype(jnp.float32)[:, None]
    blc = batch_lc.astype(jnp.float32)[:, None]

    enc_out = pl.pallas_call(
        _enc_kernel,
        in_specs=[_full((N, 6)), _full((N, 5)),
                  _full((HID, 6)), _full((HID,)), _full((HID, HID)),
                  _full((HID,)), _full((HID, 5)), _full((HID,)),
                  _full((HID, HID)), _full((HID,)),
                  _full((HID, HID)), _full((HID, HID)), _full((HID,))],
        out_specs=[_full((N, HID))] * 5 + [_full((HID, N))],
        out_shape=[jax.ShapeDtypeStruct((N, HID), jnp.float32)] * 5
        + [jax.ShapeDtypeStruct((HID, N), jnp.float32)],
        compiler_params=pltpu.CompilerParams(
            vmem_limit_bytes=60 * 1024 * 1024),
    )(x_ts, x_lc, W_ts0, b_ts0, W_ts1, b_ts1, W_lc0, b_lc0,
      W_lc1, b_lc1, A, Bm, b_conv)
    ts_enc, lc_enc, u_lc, v_lc, v_ts, lcT = enc_out

    sl, el, sn_lc = pl.pallas_call(
        _seg_kernel,
        in_specs=[_full((N, HID)), _full((N, 1))],
        out_specs=[_full((1, B)), _full((1, B)), _full((N, 1))],
        out_shape=[jax.ShapeDtypeStruct((1, B), jnp.float32),
                   jax.ShapeDtypeStruct((1, B), jnp.float32),
                   jax.ShapeDtypeStruct((N, 1), jnp.float32)],
    )(lc_enc, blc)

    lcT_f = lcT.reshape(HID * N)
    sn_lc_f = sn_lc.reshape(N)
    sl_f = sl.reshape(B)
    el_f = el.reshape(B)

    f1 = _sc_edge_conv(lcT_f, sn_lc_f, u_lc.reshape(N * HID), lc_enc.reshape(N * HID),
                       v_lc.reshape(N * HID), batch_lc, sl_f,
                       el_f).reshape(N, HID)
    f2 = _sc_edge_conv(lcT_f, sn_lc_f, u_lc.reshape(N * HID), ts_enc.reshape(N * HID),
                       v_ts.reshape(N * HID), batch_ts, sl_f,
                       el_f).reshape(N, HID)

    u_f1, v_f2, f1T, sn_f1 = pl.pallas_call(
        _proj_kernel,
        in_specs=[_full((N, HID)), _full((N, HID)),
                  _full((HID, HID)), _full((HID, HID)), _full((HID,))],
        out_specs=[_full((N, HID))] * 2 + [_full((HID, N)), _full((N, 1))],
        out_shape=[jax.ShapeDtypeStruct((N, HID), jnp.float32)] * 2
        + [jax.ShapeDtypeStruct((HID, N), jnp.float32),
           jax.ShapeDtypeStruct((N, 1), jnp.float32)],
        compiler_params=pltpu.CompilerParams(
            vmem_limit_bytes=100 * 1024 * 1024),
    )(f1, f2, A, Bm, b_conv)

    f3 = _sc_edge_conv(f1T.reshape(HID * N), sn_f1.reshape(N), u_f1.reshape(N * HID), f2.reshape(N * HID),
                       v_f2.reshape(N * HID), batch_ts, sl_f,
                       el_f).reshape(N, HID)

    out = pl.pallas_call(
        _head_kernel,
        in_specs=[_full((N, HID)), _full((N, 1)),
                  _full((64, HID)), _full((64,)), _full((32, 64)),
                  _full((32,)), _full((8, 32)), _full((8,)),
                  _full((4, 8)), _full((4,)), _full((8, 4)), _full((1,))],
        out_specs=_full((B, 1)),
        out_shape=jax.ShapeDtypeStruct((B, 1), jnp.float32),
    )(f3, bts, W_o0, b_o0, W_o1, b_o1, W_o2, b_o2, W_o3, b_o3,
      jnp.pad(W_o4, ((0, 7), (0, 0))), b_o4)

    return out, jnp.arange(B, dtype=jnp.int32)


# speedup vs baseline: 18.4776x; 1.0014x over previous
import jax
import jax.numpy as jnp
from jax import lax
from jax.experimental import pallas as pl
from jax.experimental.pallas import tpu as pltpu
from jax.experimental.pallas import tpu_sc as plsc

N = 8192
B = 16
K = 16
HID = 16
RS = 256
W = 2048
NEG = -3.0e38
PREC = jax.lax.Precision.HIGHEST


def _elu(x):
    return jnp.where(x > 0, x, jnp.exp(jnp.minimum(x, 0.0)) - 1.0)


def _enc_kernel(x_ts, x_lc, wt0, bt0, wt1, bt1, wl0, bl0, wl1, bl1,
                a_m, b_m, bc, ts_enc, lc_enc, u_lc, v_lc, v_ts, lcT):
    def lin(x, w, b):
        return jax.lax.dot_general(x, w[...], (((1,), (1,)), ((), ())),
                                   preferred_element_type=jnp.float32,
                                   precision=PREC) + b[...][None, :]
    te = _elu(lin(_elu(lin(x_ts[...], wt0, bt0)), wt1, bt1))
    le = _elu(lin(_elu(lin(x_lc[...], wl0, bl0)), wl1, bl1))
    ts_enc[...] = te
    lc_enc[...] = le
    u_lc[...] = lin(le, b_m, jnp.zeros((HID,), jnp.float32))
    v_lc[...] = lin(le, a_m, bc[...])
    v_ts[...] = lin(te, a_m, bc[...])
    eye = jnp.eye(HID, dtype=jnp.float32)
    lcT[...] = jax.lax.dot_general(eye, le, (((1,), (1,)), ((), ())),
                                   preferred_element_type=jnp.float32,
                                   precision=PREC)


def _seg_kernel(lc_enc, blc, sl, el, sn_lc):
    gid = jax.lax.broadcasted_iota(jnp.int32, (1, B), 1).astype(jnp.float32)
    oh = (blc[...] == gid).astype(jnp.float32)
    counts = jnp.sum(oh, axis=0, keepdims=True)
    r_i = jax.lax.broadcasted_iota(jnp.int32, (B, B), 0)
    c_i = jax.lax.broadcasted_iota(jnp.int32, (B, B), 1)
    strict_lower = (r_i < c_i).astype(jnp.float32)
    starts = jax.lax.dot_general(counts, strict_lower,
                                 (((1,), (0,)), ((), ())),
                                 preferred_element_type=jnp.float32,
                                 precision=PREC)
    sl[...] = starts
    el[...] = starts + counts
    le = lc_enc[...]
    sn_lc[...] = jnp.sum(le * le, axis=1, keepdims=True)


def _proj_kernel(f1, f2, a_m, b_m, bc, u_f1, v_f2, f1T, sn_f1):
    def lin(x, w, b):
        return jax.lax.dot_general(x, w[...], (((1,), (1,)), ((), ())),
                                   preferred_element_type=jnp.float32,
                                   precision=PREC) + b[...][None, :]
    u_f1[...] = lin(f1[...], b_m, jnp.zeros((HID,), jnp.float32))
    v_f2[...] = lin(f2[...], a_m, bc[...])
    eye = jnp.eye(HID, dtype=jnp.float32)
    f1T[...] = jax.lax.dot_general(eye, f1[...], (((1,), (1,)), ((), ())),
                                   preferred_element_type=jnp.float32,
                                   precision=PREC)
    sn_f1[...] = jnp.sum(f1[...] * f1[...], axis=1, keepdims=True)


def _sc_conv_body(srcT_hbm, sn_hbm, u_hbm, dst_hbm, v_hbm, bdst_hbm, sl_hbm,
                  el_hbm, out_hbm, st_win, sn_win, u_win, dstf, vf, outf,
                  bdstv, startsv, endsv, tstore, tistore, mpstore):
    wid = lax.axis_index("s") * 2 + lax.axis_index("c")
    base = wid * RS
    iota = lax.iota(jnp.int32, 16)

    pltpu.sync_copy(dst_hbm.at[pl.ds(base * HID, RS * HID)], dstf)
    pltpu.sync_copy(v_hbm.at[pl.ds(base * HID, RS * HID)], vf)
    pltpu.sync_copy(bdst_hbm.at[pl.ds(base, RS)], bdstv)
    pltpu.sync_copy(sl_hbm, startsv)
    pltpu.sync_copy(el_hbm, endsv)

    def init_body(i, _):
        tstore[pl.ds(i * 16, 16)] = jnp.full((16,), jnp.inf, jnp.float32)
        tistore[pl.ds(i * 16, 16)] = jnp.zeros((16,), jnp.int32)
        mpstore[pl.ds(i * 16, 16)] = jnp.full((16,), NEG, jnp.float32)
        return 0
    lax.fori_loop(0, RS, init_body, 0)

    svec = startsv[...].astype(jnp.int32)
    evec = endsv[...].astype(jnp.int32)

    def pick_i(vec, lane):
        return jnp.sum(jnp.where(iota == lane, vec.astype(jnp.float32),
                                 0.0)).astype(jnp.int32)

    bv0 = bdstv[pl.ds(0, 16)]
    bvl = bdstv[pl.ds(RS - 16, 16)]
    b_first = pick_i(bv0, 0)
    b_last = pick_i(bvl, 15)
    w_lo = (pick_i(svec, b_first) // 8) * 8
    w_hi = pick_i(evec, b_last)
    nwin = (w_hi - w_lo + W - 1) // W

    def win_body(wi, _):
        w0 = w_lo + wi * W
        w0c = jnp.minimum(w0, N - W)
        for f in range(HID):
            pltpu.sync_copy(srcT_hbm.at[pl.ds(f * N + w0c, W)],
                            st_win.at[pl.ds(f * W, W)])
        pltpu.sync_copy(sn_hbm.at[pl.ds(w0c, W)], sn_win)
        wcov_hi = jnp.minimum(w0 + W, N)

        def dst_body(i, _):
            xi = dstf[pl.ds(i * 16, 16)]
            bv = bdstv[pl.ds((i // 16) * 16, 16)]
            b_i = pick_i(bv, i - (i // 16) * 16)
            s_i = pick_i(svec, b_i)
            e_i = pick_i(evec, b_i)
            cs = jnp.maximum(s_i, w0)
            ce = jnp.minimum(e_i, wcov_hi)
            nch = jnp.maximum(ce - cs + 15, 0) // 16
            t = tstore[pl.ds(i * 16, 16)]
            ti = tistore[pl.ds(i * 16, 16)]
            xif = [xi[f] for f in range(HID)]
            xn = jnp.sum(xi * xi)
            t15 = jnp.max(t)

            def ch_body(c, carry):
                t, ti, t15 = carry
                j0 = cs + c * 16
                off = j0 - w0c
                prods = [st_win[pl.ds(f * W + off, 16)] * xif[f]
                         for f in range(HID)]
                while len(prods) > 1:
                    prods = [prods[a] + prods[a + 1]
                             for a in range(0, len(prods), 2)]
                cross = prods[0]
                d = (xn + sn_win[pl.ds(off, 16)]) - 2.0 * cross
                d = jnp.where(iota < (ce - j0), d, jnp.inf)
                ci = j0 + iota
                dmin = jnp.min(d)

                def mrg(tt):
                    t_, ti_ = tt
                    ds_, is_ = plsc.sort_key_val(d, ci)
                    dsr = lax.rev(ds_, (0,))
                    isr = lax.rev(is_, (0,))
                    keep = (t_ < dsr) | ((t_ == dsr) & (ti_ < isr))
                    lo_v = jnp.where(keep, t_, dsr)
                    lo_i = jnp.where(keep, ti_, isr)
                    t2, ti2 = plsc.sort_key_val(lo_v, lo_i)
                    return t2, ti2

                t, ti = lax.cond(dmin < t15, mrg, lambda tt: tt, (t, ti))
                t15 = jnp.max(t)
                return t, ti, t15

            t, ti, _ = lax.fori_loop(0, nch, ch_body, (t, ti, t15))
            tstore[pl.ds(i * 16, 16)] = t
            tistore[pl.ds(i * 16, 16)] = ti
            return 0

        lax.fori_loop(0, RS, dst_body, 0)
        return 0

    lax.fori_loop(0, nwin, win_body, 0)

    def upool_body(wi, _):
        w0 = w_lo + wi * W
        w0c = jnp.minimum(w0, N - W)
        pltpu.sync_copy(u_hbm.at[pl.ds(w0c * HID, W * HID)], u_win)

        def dst_body(i, _):
            ti = tistore[pl.ds(i * 16, 16)]
            inm = (ti >= w0) & (ti < w0 + W)
            til = jnp.where(inm, ti - w0c, 0) * HID
            mp = mpstore[pl.ds(i * 16, 16)]
            for h in range(HID):
                g = plsc.load_gather(u_win, [til + h])
                s_h = jnp.max(jnp.where(inm, g, NEG))
                mp = jnp.where(iota == h, jnp.maximum(mp, s_h), mp)
            mpstore[pl.ds(i * 16, 16)] = mp
            return 0

        lax.fori_loop(0, RS, dst_body, 0)
        return 0

    lax.fori_loop(0, nwin, upool_body, 0)

    def fin_body(i, _):
        x = vf[pl.ds(i * 16, 16)] + mpstore[pl.ds(i * 16, 16)]
        outf[pl.ds(i * 16, 16)] = jnp.where(
            x > 0, x, jnp.exp(jnp.minimum(x, 0.0)) - 1.0)
        return 0
    lax.fori_loop(0, RS, fin_body, 0)

    pltpu.sync_copy(outf, out_hbm.at[pl.ds(base * HID, RS * HID)])


def _sc_edge_conv(srcT_flat, sn_flat, u_flat, dst_flat, v_flat, bdst_i, sl_flat, el_flat):
    mesh = plsc.VectorSubcoreMesh(core_axis_name="c", subcore_axis_name="s")
    fn = pl.kernel(
        _sc_conv_body,
        out_type=pltpu.HBM((N * HID,), jnp.float32),
        mesh=mesh,
        compiler_params=pltpu.CompilerParams(needs_layout_passes=False),
        scratch_types=[
            pltpu.VMEM((HID * W,), jnp.float32),
            pltpu.VMEM((W,), jnp.float32),
            pltpu.VMEM((W * HID,), jnp.float32),
            pltpu.VMEM((RS * HID,), jnp.float32),
            pltpu.VMEM((RS * HID,), jnp.float32),
            pltpu.VMEM((RS * HID,), jnp.float32),
            pltpu.VMEM((RS,), jnp.int32),
            pltpu.VMEM((B,), jnp.float32),
            pltpu.VMEM((B,), jnp.float32),
            pltpu.VMEM((RS * 16,), jnp.float32),
            pltpu.VMEM((RS * 16,), jnp.int32),
            pltpu.VMEM((RS * 16,), jnp.float32),
        ],
    )
    return fn(srcT_flat, sn_flat, u_flat, dst_flat, v_flat, bdst_i, sl_flat, el_flat)


def _head_kernel(f3, bts, w0, b0, w1, b1, w2, b2, w3, b3, w4, b4, out):
    def lin(x, w, b):
        return jax.lax.dot_general(x, w[...], (((1,), (1,)), ((), ())),
                                   preferred_element_type=jnp.float32,
                                   precision=PREC) + b[...][None, :]
    gid = jax.lax.broadcasted_iota(jnp.int32, (1, B), 1).astype(jnp.float32)
    oh = (bts[...] == gid).astype(jnp.float32)
    pooled = jax.lax.dot_general(oh, f3[...], (((0,), (0,)), ((), ())),
                                 preferred_element_type=jnp.float32,
                                 precision=PREC)
    cnt = jnp.sum(oh, axis=0)[:, None]
    pooled = pooled / jnp.maximum(cnt, 1.0)
    h = _elu(lin(pooled, w0, b0))
    h = _elu(lin(h, w1, b1))
    h = _elu(lin(h, w2, b2))
    h = _elu(lin(h, w3, b3))
    o8 = jax.lax.dot_general(h, w4[...], (((1,), (1,)), ((), ())),
                             preferred_element_type=jnp.float32,
                             precision=PREC) + b4[0]
    out[...] = o8[:, 0:1]


def _full(shape):
    return pl.BlockSpec(shape, lambda *_: tuple(0 for _ in shape))


def kernel(x_ts, x_lc, batch_ts, batch_lc, W_ts0, b_ts0, W_ts1, b_ts1,
           W_lc0, b_lc0, W_lc1, b_lc1, W_conv, b_conv, W_o0, b_o0,
           W_o1, b_o1, W_o2, b_o2, W_o3, b_o3, W_o4, b_o4):
    A = W_conv[:, :HID] - W_conv[:, HID:]
    Bm = W_conv[:, HID:]
    bts = batch_ts.astype(jnp.float32)[:, None]
    blc = batch_lc.astype(jnp.float32)[:, None]

    enc_out = pl.pallas_call(
        _enc_kernel,
        in_specs=[_full((N, 6)), _full((N, 5)),
                  _full((HID, 6)), _full((HID,)), _full((HID, HID)),
                  _full((HID,)), _full((HID, 5)), _full((HID,)),
                  _full((HID, HID)), _full((HID,)),
                  _full((HID, HID)), _full((HID, HID)), _full((HID,))],
        out_specs=[_full((N, HID))] * 5 + [_full((HID, N))],
        out_shape=[jax.ShapeDtypeStruct((N, HID), jnp.float32)] * 5
        + [jax.ShapeDtypeStruct((HID, N), jnp.float32)],
        compiler_params=pltpu.CompilerParams(
            vmem_limit_bytes=60 * 1024 * 1024),
    )(x_ts, x_lc, W_ts0, b_ts0, W_ts1, b_ts1, W_lc0, b_lc0,
      W_lc1, b_lc1, A, Bm, b_conv)
    ts_enc, lc_enc, u_lc, v_lc, v_ts, lcT = enc_out

    sl, el, sn_lc = pl.pallas_call(
        _seg_kernel,
        in_specs=[_full((N, HID)), _full((N, 1))],
        out_specs=[_full((1, B)), _full((1, B)), _full((N, 1))],
        out_shape=[jax.ShapeDtypeStruct((1, B), jnp.float32),
                   jax.ShapeDtypeStruct((1, B), jnp.float32),
                   jax.ShapeDtypeStruct((N, 1), jnp.float32)],
    )(lc_enc, blc)

    lcT_f = lcT.reshape(HID * N)
    sn_lc_f = sn_lc.reshape(N)
    sl_f = sl.reshape(B)
    el_f = el.reshape(B)

    f1 = _sc_edge_conv(lcT_f, sn_lc_f, u_lc.reshape(N * HID), lc_enc.reshape(N * HID),
                       v_lc.reshape(N * HID), batch_lc, sl_f,
                       el_f).reshape(N, HID)
    f2 = _sc_edge_conv(lcT_f, sn_lc_f, u_lc.reshape(N * HID), ts_enc.reshape(N * HID),
                       v_ts.reshape(N * HID), batch_ts, sl_f,
                       el_f).reshape(N, HID)

    u_f1, v_f2, f1T, sn_f1 = pl.pallas_call(
        _proj_kernel,
        in_specs=[_full((N, HID)), _full((N, HID)),
                  _full((HID, HID)), _full((HID, HID)), _full((HID,))],
        out_specs=[_full((N, HID))] * 2 + [_full((HID, N)), _full((N, 1))],
        out_shape=[jax.ShapeDtypeStruct((N, HID), jnp.float32)] * 2
        + [jax.ShapeDtypeStruct((HID, N), jnp.float32),
           jax.ShapeDtypeStruct((N, 1), jnp.float32)],
        compiler_params=pltpu.CompilerParams(
            vmem_limit_bytes=100 * 1024 * 1024),
    )(f1, f2, A, Bm, b_conv)

    f3 = _sc_edge_conv(f1T.reshape(HID * N), sn_f1.reshape(N), u_f1.reshape(N * HID), f2.reshape(N * HID),
                       v_f2.reshape(N * HID), batch_ts, sl_f,
                       el_f).reshape(N, HID)

    out = pl.pallas_call(
        _head_kernel,
        in_specs=[_full((N, HID)), _full((N, 1)),
                  _full((64, HID)), _full((64,)), _full((32, 64)),
                  _full((32,)), _full((8, 32)), _full((8,)),
                  _full((4, 8)), _full((4,)), _full((8, 4)), _full((1,))],
        out_specs=_full((B, 1)),
        out_shape=jax.ShapeDtypeStruct((B, 1), jnp.float32),
    )(f3, bts, W_o0, b_o0, W_o1, b_o1, W_o2, b_o2, W_o3, b_o3,
      jnp.pad(W_o4, ((0, 7), (0, 0))), b_o4)

    return out, jnp.arange(B, dtype=jnp.int32)
